# baseline (speedup 1.0000x reference)
import functools

import jax
import jax.numpy as jnp
from jax import lax
from jax.experimental import pallas as pl
from jax.experimental.pallas import tpu as pltpu
from jax.experimental.pallas import tpu_sc as plsc

_RB = 64
_K = 3


def _pooled_body(l, v, ids_ref, seg_ref, emb_ref, W_ref, b_ref, pooled_ref):
    ids = ids_ref[0]
    tb = ids.shape[1]
    iota_v = lax.broadcasted_iota(jnp.int32, (v, tb), 0)
    onehot_t = (iota_v == ids).astype(jnp.float32)
    counts_t = lax.dot_general(
        onehot_t, seg_ref[...], (((1,), (0,)), ((), ())),
        preferred_element_type=jnp.float32)
    pooled = lax.dot_general(
        counts_t, emb_ref[...], (((0,), (0,)), ((), ())),
        preferred_element_type=jnp.float32) * (1.0 / l)
    pooled = lax.dot_general(
        pooled, W_ref[...], (((1,), (1,)), ((), ())),
        preferred_element_type=jnp.float32)
    pooled_ref[...] = pooled + b_ref[...]


def _make_sc_gather(n_rows, lanes, h, n_workers, v):
    rows_per_w = n_rows // n_workers
    mesh = plsc.VectorSubcoreMesh(core_axis_name="c", subcore_axis_name="s")

    def body(ids_hbm, emb_hbm, out_hbm, idx_v, table_v, rows0, rows1,
             gs0, gs1, ws0, ws1):
        sid = lax.axis_index("s")
        wid = sid * plsc.get_sparse_core_info().num_cores \
            + lax.axis_index("c")
        base = wid * rows_per_w

        @pl.when(sid == 0)
        def _stage_table():
            pltpu.sync_copy(emb_hbm, table_v)

        plsc.subcore_barrier()
        pltpu.sync_copy(ids_hbm.at[pl.ds(base, rows_per_w)], idx_v)
        nfull = rows_per_w // _K
        rem = rows_per_w - nfull * _K
        npairs = nfull // 2

        def fire_g(rows_v, sem, c):
            row0 = c * _K
            for j in range(_K):
                pltpu.async_copy(
                    table_v.at[idx_v.at[row0 + j]], rows_v.at[j], sem)

        def drain_g(rows_v, sem, k=_K):
            pltpu.make_async_copy(
                out_hbm.at[pl.ds(base, k)], rows_v.at[pl.ds(0, k)],
                sem).wait()

        def fire_w(rows_v, sem, c, k=_K):
            pltpu.make_async_copy(
                rows_v.at[pl.ds(0, k)],
                out_hbm.at[pl.ds(base + c * _K, k)], sem).start()

        def drain_w(rows_v, sem, k=_K):
            pltpu.make_async_copy(
                rows_v.at[pl.ds(0, k)], out_hbm.at[pl.ds(base, k)],
                sem).wait()

        fire_g(rows0, gs0, 0)
        fire_g(rows1, gs1, 1)

        def pair(p, carry):
            c0 = 2 * p
            drain_g(rows0, gs0)
            fire_w(rows0, ws0, c0)
            drain_g(rows1, gs1)
            fire_w(rows1, ws1, c0 + 1)
            drain_w(rows0, ws0)
            fire_g(rows0, gs0, c0 + 2)
            drain_w(rows1, ws1)
            fire_g(rows1, gs1, c0 + 3)
            return carry

        lax.fori_loop(0, npairs - 1, pair, 0)

        c0 = 2 * (npairs - 1)
        drain_g(rows0, gs0)
        fire_w(rows0, ws0, c0)
        drain_g(rows1, gs1)
        fire_w(rows1, ws1, c0 + 1)
        drain_w(rows0, ws0)
        if rem:
            row0 = nfull * _K
            for j in range(rem):
                pltpu.async_copy(
                    table_v.at[idx_v.at[row0 + j]], rows0.at[j], gs0)
            drain_g(rows0, gs0, rem)
            pltpu.make_async_copy(
                rows0.at[pl.ds(0, rem)],
                out_hbm.at[pl.ds(base + row0, rem)], ws0).start()
            drain_w(rows0, ws0, rem)
        drain_w(rows1, ws1)

    return pl.kernel(
        body,
        mesh=mesh,
        out_type=jax.ShapeDtypeStruct((n_rows, lanes, h), jnp.float32),
        scratch_types=[
            pltpu.VMEM((rows_per_w, lanes), jnp.int32),
            pltpu.VMEM_SHARED((v, h), jnp.float32),
            pltpu.VMEM((_K, lanes, h), jnp.float32),
            pltpu.VMEM((_K, lanes, h), jnp.float32),
            pltpu.SemaphoreType.DMA,
            pltpu.SemaphoreType.DMA,
            pltpu.SemaphoreType.DMA,
            pltpu.SemaphoreType.DMA,
        ],
    )


@jax.jit
def kernel(input_ids, attention_mask, emb, W, b):
    del attention_mask
    ids = input_ids.astype(jnp.int32)
    bsz, l = ids.shape
    v, h = emb.shape
    lanes = 128
    n_rows = (bsz * l) // lanes
    info = plsc.get_sparse_core_info()
    n_workers = info.num_cores * info.num_subcores

    ids2 = ids.reshape(n_rows, lanes)
    embeds2 = _make_sc_gather(n_rows, lanes, h, n_workers, v)(ids2, emb)

    nb = bsz // _RB
    tb = _RB * l
    ids3 = ids.reshape(nb, 1, tb)
    seg = (jax.lax.broadcasted_iota(jnp.int32, (tb, _RB), 0) // l ==
           jax.lax.broadcasted_iota(jnp.int32, (tb, _RB), 1)
           ).astype(jnp.float32)
    pooled = pl.pallas_call(
        functools.partial(_pooled_body, l, v),
        grid=(nb,),
        in_specs=[
            pl.BlockSpec((1, 1, tb), lambda i: (i, 0, 0)),
            pl.BlockSpec((tb, _RB), lambda i: (0, 0)),
            pl.BlockSpec((v, h), lambda i: (0, 0)),
            pl.BlockSpec((h, h), lambda i: (0, 0)),
            pl.BlockSpec((1, h), lambda i: (0, 0)),
        ],
        out_specs=pl.BlockSpec((_RB, h), lambda i: (i, 0)),
        out_shape=jax.ShapeDtypeStruct((bsz, h), jnp.float32),
    )(ids3, seg, emb, W, b.reshape(1, h))

    return (pooled, embeds2.reshape(bsz, l, h))


# speedup vs baseline: 1.0128x; 1.0128x over previous
import functools

import jax
import jax.numpy as jnp
from jax import lax
from jax.experimental import pallas as pl
from jax.experimental.pallas import tpu as pltpu
from jax.experimental.pallas import tpu_sc as plsc

_RB = 64
_K = 3


def _pooled_body(l, v, ids_ref, emb_ref, W_ref, b_ref, pooled_ref):
    ids = ids_ref[0]
    tb = ids.shape[1]
    rb = tb // l
    iota_v = lax.broadcasted_iota(jnp.int32, (v, tb), 0)
    onehot_t = (iota_v == ids).astype(jnp.float32)
    t_iota = lax.broadcasted_iota(jnp.int32, (tb, rb), 0)
    r_iota = lax.broadcasted_iota(jnp.int32, (tb, rb), 1)
    seg = (t_iota // l == r_iota).astype(jnp.float32)
    counts_t = lax.dot_general(
        onehot_t, seg, (((1,), (0,)), ((), ())),
        preferred_element_type=jnp.float32)
    pooled = lax.dot_general(
        counts_t, emb_ref[...], (((0,), (0,)), ((), ())),
        preferred_element_type=jnp.float32) * (1.0 / l)
    pooled = lax.dot_general(
        pooled, W_ref[...], (((1,), (1,)), ((), ())),
        preferred_element_type=jnp.float32)
    pooled_ref[...] = pooled + b_ref[...]


def _make_sc_gather(n_rows, lanes, h, n_workers, v):
    rows_per_w = n_rows // n_workers
    mesh = plsc.VectorSubcoreMesh(core_axis_name="c", subcore_axis_name="s")

    def body(ids_hbm, emb_hbm, out_hbm, idx_v, table_v, rows0, rows1,
             gs0, gs1, ws0, ws1):
        sid = lax.axis_index("s")
        wid = sid * plsc.get_sparse_core_info().num_cores \
            + lax.axis_index("c")
        base = wid * rows_per_w

        @pl.when(sid == 0)
        def _stage_table():
            pltpu.sync_copy(emb_hbm, table_v)

        plsc.subcore_barrier()
        pltpu.sync_copy(ids_hbm.at[pl.ds(base, rows_per_w)], idx_v)
        nfull = rows_per_w // _K
        rem = rows_per_w - nfull * _K
        npairs = nfull // 2

        def fire_g(rows_v, sem, c):
            row0 = c * _K
            for j in range(_K):
                pltpu.async_copy(
                    table_v.at[idx_v.at[row0 + j]], rows_v.at[j], sem)

        def drain_g(rows_v, sem, k=_K):
            pltpu.make_async_copy(
                out_hbm.at[pl.ds(base, k)], rows_v.at[pl.ds(0, k)],
                sem).wait()

        def fire_w(rows_v, sem, c, k=_K):
            pltpu.make_async_copy(
                rows_v.at[pl.ds(0, k)],
                out_hbm.at[pl.ds(base + c * _K, k)], sem).start()

        def drain_w(rows_v, sem, k=_K):
            pltpu.make_async_copy(
                rows_v.at[pl.ds(0, k)], out_hbm.at[pl.ds(base, k)],
                sem).wait()

        fire_g(rows0, gs0, 0)
        fire_g(rows1, gs1, 1)

        def pair(p, carry):
            c0 = 2 * p
            drain_g(rows0, gs0)
            fire_w(rows0, ws0, c0)
            drain_g(rows1, gs1)
            fire_w(rows1, ws1, c0 + 1)
            drain_w(rows0, ws0)
            fire_g(rows0, gs0, c0 + 2)
            drain_w(rows1, ws1)
            fire_g(rows1, gs1, c0 + 3)
            return carry

        lax.fori_loop(0, npairs - 1, pair, 0)

        c0 = 2 * (npairs - 1)
        drain_g(rows0, gs0)
        fire_w(rows0, ws0, c0)
        drain_g(rows1, gs1)
        fire_w(rows1, ws1, c0 + 1)
        drain_w(rows0, ws0)
        if rem:
            row0 = nfull * _K
            for j in range(rem):
                pltpu.async_copy(
                    table_v.at[idx_v.at[row0 + j]], rows0.at[j], gs0)
            drain_g(rows0, gs0, rem)
            pltpu.make_async_copy(
                rows0.at[pl.ds(0, rem)],
                out_hbm.at[pl.ds(base + row0, rem)], ws0).start()
            drain_w(rows0, ws0, rem)
        drain_w(rows1, ws1)

    return pl.kernel(
        body,
        mesh=mesh,
        out_type=jax.ShapeDtypeStruct((n_rows, lanes, h), jnp.float32),
        scratch_types=[
            pltpu.VMEM((rows_per_w, lanes), jnp.int32),
            pltpu.VMEM_SHARED((v, h), jnp.float32),
            pltpu.VMEM((_K, lanes, h), jnp.float32),
            pltpu.VMEM((_K, lanes, h), jnp.float32),
            pltpu.SemaphoreType.DMA,
            pltpu.SemaphoreType.DMA,
            pltpu.SemaphoreType.DMA,
            pltpu.SemaphoreType.DMA,
        ],
    )


@jax.jit
def kernel(input_ids, attention_mask, emb, W, b):
    del attention_mask
    ids = input_ids.astype(jnp.int32)
    bsz, l = ids.shape
    v, h = emb.shape
    lanes = 128
    n_rows = (bsz * l) // lanes
    info = plsc.get_sparse_core_info()
    n_workers = info.num_cores * info.num_subcores

    ids2 = ids.reshape(n_rows, lanes)
    embeds2 = _make_sc_gather(n_rows, lanes, h, n_workers, v)(ids2, emb)

    nb = bsz // _RB
    tb = _RB * l
    ids3 = ids.reshape(nb, 1, tb)
    pooled = pl.pallas_call(
        functools.partial(_pooled_body, l, v),
        grid=(nb,),
        in_specs=[
            pl.BlockSpec((1, 1, tb), lambda i: (i, 0, 0)),
            pl.BlockSpec((v, h), lambda i: (0, 0)),
            pl.BlockSpec((h, h), lambda i: (0, 0)),
            pl.BlockSpec((1, h), lambda i: (0, 0)),
        ],
        out_specs=pl.BlockSpec((_RB, h), lambda i: (i, 0)),
        out_shape=jax.ShapeDtypeStruct((bsz, h), jnp.float32),
    )(ids3, emb, W, b.reshape(1, h))

    return (pooled, embeds2.reshape(bsz, l, h))
